# baseline (speedup 1.0000x reference)
import jax
import jax.numpy as jnp
from jax import lax
from jax.experimental import pallas as pl
from jax.experimental.pallas import tpu as pltpu

N_EDGES = 3_200_000
N_NODES = 100_000
D_EDGE = 16

EGRID = 25
EBLK = N_EDGES // EGRID
EACC_W = 3200

NGRID = 25
NBLK = N_NODES // NGRID


def _edges_body(edges_ref, esum_ref, eacc):
    g = pl.program_id(0)

    @pl.when(g == 0)
    def _init():
        eacc[...] = jnp.zeros_like(eacc)

    e = eacc[...]
    for s in range(EBLK // EACC_W):
        e = e + edges_ref[:, pl.ds(s * EACC_W, EACC_W)]
    eacc[...] = e

    @pl.when(g == EGRID - 1)
    def _fin():
        esum_ref[...] = eacc[...]


def _nodes_fin_body(glob_ref, nodes_ref, esum_ref, WgT_ref, WeT_ref,
                    WnT_ref, b_ref, out_ref, nacc):
    g = pl.program_id(0)

    @pl.when(g == 0)
    def _init():
        nacc[...] = jnp.zeros_like(nacc)

    nacc[...] += jnp.sum(nodes_ref[...], axis=0, keepdims=True)

    @pl.when(g == NGRID - 1)
    def _fin():
        erow = jnp.dot(esum_ref[...], jnp.ones((EACC_W, 1), jnp.float32),
                       preferred_element_type=jnp.float32)
        e_out = lax.dot_general(
            erow, WeT_ref[...], (((0,), (0,)), ((), ())),
            preferred_element_type=jnp.float32)
        n_row = nacc[...] * (1.0 / N_NODES)
        out_ref[...] = (
            jnp.dot(glob_ref[...], WgT_ref[...],
                    preferred_element_type=jnp.float32)
            + e_out * (1.0 / N_EDGES)
            + jnp.dot(n_row, WnT_ref[...],
                      preferred_element_type=jnp.float32)
            + b_ref[...])


def kernel(global_data, nodes_data, edges_data, W, b):
    edges_t = edges_data.T
    esum = pl.pallas_call(
        _edges_body,
        grid=(EGRID,),
        in_specs=[pl.BlockSpec((D_EDGE, EBLK), lambda g: (0, g))],
        out_specs=pl.BlockSpec((D_EDGE, EACC_W), lambda g: (0, 0)),
        out_shape=jax.ShapeDtypeStruct((D_EDGE, EACC_W), jnp.float32),
        scratch_shapes=[pltpu.VMEM((D_EDGE, EACC_W), jnp.float32)],
    )(edges_t)
    WT = W.T
    out = pl.pallas_call(
        _nodes_fin_body,
        grid=(NGRID,),
        in_specs=[
            pl.BlockSpec((1, 128), lambda g: (0, 0)),
            pl.BlockSpec((NBLK, 128), lambda g: (g, 0)),
            pl.BlockSpec((D_EDGE, EACC_W), lambda g: (0, 0)),
            pl.BlockSpec((128, 128), lambda g: (0, 0)),
            pl.BlockSpec((16, 128), lambda g: (0, 0)),
            pl.BlockSpec((128, 128), lambda g: (0, 0)),
            pl.BlockSpec((1, 128), lambda g: (0, 0)),
        ],
        out_specs=pl.BlockSpec((1, 128), lambda g: (0, 0)),
        out_shape=jax.ShapeDtypeStruct((1, 128), jnp.float32),
        scratch_shapes=[pltpu.VMEM((1, 128), jnp.float32)],
    )(global_data[None, :], nodes_data, esum, WT[:128], WT[128:144],
      WT[144:], b[None, :])
    return out[0]


# speedup vs baseline: 1.1767x; 1.1767x over previous
import jax
import jax.numpy as jnp
from jax import lax
from jax.experimental import pallas as pl
from jax.experimental.pallas import tpu as pltpu

N_EDGES = 3_200_000
N_NODES = 100_000
D_EDGE = 16

GRID = 25
EBLK = N_EDGES // GRID
EACC_W = 3200
NBLK = N_NODES // GRID


def _body(glob_ref, nodes_ref, edges_ref, WgT_ref, WeT_ref, WnT_ref, b_ref,
          out_ref, nacc, eacc):
    g = pl.program_id(0)

    @pl.when(g == 0)
    def _init():
        nacc[...] = jnp.zeros_like(nacc)
        eacc[...] = jnp.zeros_like(eacc)

    nacc[...] += jnp.sum(nodes_ref[...], axis=0, keepdims=True)
    e = eacc[...]
    for s in range(EBLK // EACC_W):
        e = e + edges_ref[:, pl.ds(s * EACC_W, EACC_W)]
    eacc[...] = e

    @pl.when(g == GRID - 1)
    def _fin():
        erow = jnp.dot(eacc[...], jnp.ones((EACC_W, 1), jnp.float32),
                       preferred_element_type=jnp.float32)
        e_out = lax.dot_general(
            erow, WeT_ref[...], (((0,), (0,)), ((), ())),
            preferred_element_type=jnp.float32)
        n_row = nacc[...] * (1.0 / N_NODES)
        out_ref[...] = (
            jnp.dot(glob_ref[...], WgT_ref[...],
                    preferred_element_type=jnp.float32)
            + e_out * (1.0 / N_EDGES)
            + jnp.dot(n_row, WnT_ref[...], preferred_element_type=jnp.float32)
            + b_ref[...])


def kernel(global_data, nodes_data, edges_data, W, b):
    edges_t = edges_data.T
    WT = W.T
    out = pl.pallas_call(
        _body,
        grid=(GRID,),
        in_specs=[
            pl.BlockSpec((1, 128), lambda g: (0, 0)),
            pl.BlockSpec((NBLK, 128), lambda g: (g, 0)),
            pl.BlockSpec((D_EDGE, EBLK), lambda g: (0, g)),
            pl.BlockSpec((128, 128), lambda g: (0, 0)),
            pl.BlockSpec((16, 128), lambda g: (0, 0)),
            pl.BlockSpec((128, 128), lambda g: (0, 0)),
            pl.BlockSpec((1, 128), lambda g: (0, 0)),
        ],
        out_specs=pl.BlockSpec((1, 128), lambda g: (0, 0)),
        out_shape=jax.ShapeDtypeStruct((1, 128), jnp.float32),
        scratch_shapes=[
            pltpu.VMEM((1, 128), jnp.float32),
            pltpu.VMEM((D_EDGE, EACC_W), jnp.float32),
        ],
    )(global_data[None, :], nodes_data, edges_t, WT[:128], WT[128:144],
      WT[144:], b[None, :])
    return out[0]
